# baseline (speedup 1.0000x reference)
import jax
import jax.numpy as jnp
from jax import lax
from jax.experimental import pallas as pl
from jax.experimental.pallas import tpu as pltpu
from jax.experimental.pallas import tpu_sc as plsc


def _fern_sc(T, iparams, fparams, N, NB, Hout, Wout):
    H, W = T.shape[2], T.shape[3]
    ROWS_HALF = 105
    CHUNK = 35
    IMG_ROWS = 120

    mesh = plsc.VectorSubcoreMesh(core_axis_name="c", subcore_axis_name="s")

    @jax.named_call
    def run(T, iparams, fparams):
        @pl.kernel(
            out_type=jax.ShapeDtypeStruct((N, NB, Hout, Wout), jnp.float32),
            mesh=mesh,
            compiler_params=pltpu.CompilerParams(use_tc_tiling_on_sc=False),
            scratch_types=[
                pltpu.VMEM((3, IMG_ROWS, W), jnp.float32),
                pltpu.VMEM((NB, 16), jnp.int32),
                pltpu.VMEM((NB, 16), jnp.float32),
                pltpu.VMEM((CHUNK, Wout), jnp.float32),
            ],
        )
        def k(T_hbm, ipar_hbm, fpar_hbm, out_hbm, img_v, ipar_v, fpar_v, obuf_v):
            wid = lax.axis_index("s") * 2 + lax.axis_index("c")
            n = wid >> 1
            half = wid & 1
            y0 = half * 104

            pltpu.sync_copy(ipar_hbm, ipar_v)
            pltpu.sync_copy(fpar_hbm, fpar_v)
            for c in range(3):
                pltpu.sync_copy(T_hbm.at[n, c, pl.ds(y0, IMG_ROWS)], img_v.at[c])

            def bit_body(b, carry):
                ivec = ipar_v[b]
                fvec = fpar_v[b]
                iy1 = ivec[0]
                ix1 = ivec[1]
                iy2 = ivec[2]
                ix2 = ivec[3]
                ch = ivec[4]
                w = [jnp.full((16,), fvec[j], jnp.float32) for j in range(8)]
                thv = jnp.full((16,), fvec[8], jnp.float32)
                apos = jnp.full((16,), fvec[9], jnp.float32)
                aneg = jnp.full((16,), fvec[10], jnp.float32)

                for c3 in range(3):
                    def row_body(rc, cr):
                        r = c3 * CHUNK + rc
                        ra = r + iy1
                        rb = r + iy2
                        for j in range(14):
                            x0 = 193 if j == 13 else j * 16
                            a00 = img_v[ch, ra, pl.ds(ix1 + x0, 16)]
                            a01 = img_v[ch, ra, pl.ds(ix1 + x0 + 1, 16)]
                            a10 = img_v[ch, ra + 1, pl.ds(ix1 + x0, 16)]
                            a11 = img_v[ch, ra + 1, pl.ds(ix1 + x0 + 1, 16)]
                            b00 = img_v[ch, rb, pl.ds(ix2 + x0, 16)]
                            b01 = img_v[ch, rb, pl.ds(ix2 + x0 + 1, 16)]
                            b10 = img_v[ch, rb + 1, pl.ds(ix2 + x0, 16)]
                            b11 = img_v[ch, rb + 1, pl.ds(ix2 + x0 + 1, 16)]
                            p1 = (w[0] * a00 + w[1] * a01) + (w[2] * a10 + w[3] * a11)
                            p2 = (w[4] * b00 + w[5] * b01) + (w[6] * b10 + w[7] * b11)
                            d = (p1 - p2) - thv
                            s = jnp.where(d >= 0.0, d * apos, d * aneg)
                            res = jnp.minimum(jnp.maximum(s + 0.5, 0.0), 1.0)
                            obuf_v[rc, pl.ds(x0, 16)] = res
                        return cr
                    lax.fori_loop(0, CHUNK, row_body, 0)
                    pltpu.sync_copy(
                        obuf_v, out_hbm.at[n, b, pl.ds(y0 + c3 * CHUNK, CHUNK)])
                return carry

            lax.fori_loop(0, NB, bit_body, 0)

        return k(T, iparams, fparams)

    return run(T, iparams, fparams)


def kernel(T, dx1, dx2, dy1, dy2, th, amb, channels):
    N, Dc, H, W = T.shape
    M_, K_ = dx1.shape
    NB = M_ * K_
    P_ = 16
    Hout = H - P_ + 1
    Wout = W - P_ + 1
    pm = (P_ - 1) // 2

    def taps(dy, dx):
        fly, flx = jnp.floor(dy), jnp.floor(dx)
        iy = (pm + fly).astype(jnp.int32)
        ix = (pm + flx).astype(jnp.int32)
        fy = jnp.abs(dy - fly)
        fx = jnp.abs(dx - flx)
        w00 = (1.0 - fy) * (1.0 - fx)
        w01 = (1.0 - fy) * fx
        w10 = fy * (1.0 - fx)
        w11 = fy * fx
        return iy, ix, w00, w01, w10, w11

    iy1, ix1, w100, w101, w110, w111 = taps(dy1, dx1)
    iy2, ix2, w200, w201, w210, w211 = taps(dy2, dx2)
    apos = 0.5 / amb[:, 0, :]
    aneg = 0.5 / amb[:, 1, :]
    chb = jnp.broadcast_to(channels[None, :], (M_, K_))

    zi = jnp.zeros((NB,), jnp.int32)
    iparams = jnp.stack(
        [iy1.ravel(), ix1.ravel(), iy2.ravel(), ix2.ravel(), chb.ravel()]
        + [zi] * 11, axis=1)
    zf = jnp.zeros((NB,), jnp.float32)
    fparams = jnp.stack(
        [w100.ravel(), w101.ravel(), w110.ravel(), w111.ravel(),
         w200.ravel(), w201.ravel(), w210.ravel(), w211.ravel(),
         th.ravel(), apos.ravel(), aneg.ravel()] + [zf] * 5, axis=1)

    return _fern_sc(T, iparams, fparams, N, NB, Hout, Wout)


# speedup vs baseline: 1.0637x; 1.0637x over previous
import jax
import jax.numpy as jnp
from jax import lax
from jax.experimental import pallas as pl
from jax.experimental.pallas import tpu as pltpu
from jax.experimental.pallas import tpu_sc as plsc


def _fern_sc(Tf, iparams, fparams, N, NB, Hout, Wout, W):
    ROWS_HALF = 105
    IMG_ROWS = 121
    CH_STRIDE = IMG_ROWS * W
    LOAD_ROWS = 120

    mesh = plsc.VectorSubcoreMesh(core_axis_name="c", subcore_axis_name="s")

    @pl.kernel(
        out_type=jax.ShapeDtypeStruct((N, NB, Hout, Wout), jnp.float32),
        mesh=mesh,
        compiler_params=pltpu.CompilerParams(
            use_tc_tiling_on_sc=False, needs_layout_passes=False),
        scratch_types=[
            pltpu.VMEM((3 * CH_STRIDE,), jnp.float32),
            pltpu.VMEM((NB, 16), jnp.int32),
            pltpu.VMEM((NB, 16), jnp.float32),
            pltpu.VMEM((ROWS_HALF, Wout), jnp.float32),
        ],
    )
    def k(T_hbm, ipar_hbm, fpar_hbm, out_hbm, img_v, ipar_v, fpar_v, obuf_v):
        wid = lax.axis_index("s") * 2 + lax.axis_index("c")
        n = wid >> 1
        half = wid & 1
        y0 = half * 104

        pltpu.sync_copy(ipar_hbm, ipar_v)
        pltpu.sync_copy(fpar_hbm, fpar_v)
        for c in range(3):
            pltpu.sync_copy(
                T_hbm.at[n, c, pl.ds(y0 * W, LOAD_ROWS * W)],
                img_v.at[pl.ds(c * CH_STRIDE, LOAD_ROWS * W)])

        iota = lax.iota(jnp.int32, 16)

        def bit_body(b, carry):
            ivec = ipar_v[b]
            fvec = fpar_v[b]
            off1 = ivec[0]
            off2 = ivec[1]
            w0 = jnp.full((16,), fvec[0], jnp.float32)
            w1 = jnp.full((16,), fvec[1], jnp.float32)
            w2 = jnp.full((16,), fvec[2], jnp.float32)
            w3 = jnp.full((16,), fvec[3], jnp.float32)
            w4 = jnp.full((16,), fvec[4], jnp.float32)
            w5 = jnp.full((16,), fvec[5], jnp.float32)
            w6 = jnp.full((16,), fvec[6], jnp.float32)
            w7 = jnp.full((16,), fvec[7], jnp.float32)
            thv = jnp.full((16,), fvec[8], jnp.float32)
            apos = jnp.full((16,), fvec[9], jnp.float32)
            aneg = jnp.full((16,), fvec[10], jnp.float32)

            for j in range(14):
                x0 = 193 if j == 13 else 16 * j
                idxa = iota + (off1 + x0)
                idxb = iota + (off2 + x0)
                a00 = plsc.load_gather(img_v, [idxa])
                a01 = plsc.load_gather(img_v, [idxa + 1])
                b00 = plsc.load_gather(img_v, [idxb])
                b01 = plsc.load_gather(img_v, [idxb + 1])
                idxa = idxa + W
                idxb = idxb + W
                a10 = plsc.load_gather(img_v, [idxa])
                a11 = plsc.load_gather(img_v, [idxa + 1])
                b10 = plsc.load_gather(img_v, [idxb])
                b11 = plsc.load_gather(img_v, [idxb + 1])

                def row_body(r, cr):
                    (a00, a01, a10, a11, b00, b01, b10, b11, idxa, idxb) = cr
                    p1 = (w0 * a00 + w1 * a01) + (w2 * a10 + w3 * a11)
                    p2 = (w4 * b00 + w5 * b01) + (w6 * b10 + w7 * b11)
                    d = (p1 - p2) - thv
                    asel = jnp.where(d >= 0.0, apos, aneg)
                    res = jnp.minimum(
                        jnp.maximum(d * asel + 0.5, 0.0), 1.0)
                    obuf_v[r, pl.ds(x0, 16)] = res
                    ia = idxa + W
                    ib = idxb + W
                    na0 = plsc.load_gather(img_v, [ia])
                    na1 = plsc.load_gather(img_v, [ia + 1])
                    nb0 = plsc.load_gather(img_v, [ib])
                    nb1 = plsc.load_gather(img_v, [ib + 1])
                    return (a10, a11, na0, na1, b10, b11, nb0, nb1, ia, ib)

                lax.fori_loop(
                    0, ROWS_HALF, row_body,
                    (a00, a01, a10, a11, b00, b01, b10, b11, idxa, idxb),
                    unroll=5)

            pltpu.sync_copy(obuf_v, out_hbm.at[n, b, pl.ds(y0, ROWS_HALF)])
            return carry

        lax.fori_loop(0, NB, bit_body, 0)

    return k(Tf, iparams, fparams)


def kernel(T, dx1, dx2, dy1, dy2, th, amb, channels):
    N, Dc, H, W = T.shape
    M_, K_ = dx1.shape
    NB = M_ * K_
    P_ = 16
    Hout = H - P_ + 1
    Wout = W - P_ + 1
    pm = (P_ - 1) // 2
    ch_stride = (104 + P_ + 1) * W

    def taps(dy, dx):
        fly, flx = jnp.floor(dy), jnp.floor(dx)
        iy = (pm + fly).astype(jnp.int32)
        ix = (pm + flx).astype(jnp.int32)
        fy = jnp.abs(dy - fly)
        fx = jnp.abs(dx - flx)
        w00 = (1.0 - fy) * (1.0 - fx)
        w01 = (1.0 - fy) * fx
        w10 = fy * (1.0 - fx)
        w11 = fy * fx
        return iy, ix, w00, w01, w10, w11

    iy1, ix1, w100, w101, w110, w111 = taps(dy1, dx1)
    iy2, ix2, w200, w201, w210, w211 = taps(dy2, dx2)
    apos = 0.5 / amb[:, 0, :]
    aneg = 0.5 / amb[:, 1, :]
    chb = jnp.broadcast_to(channels[None, :], (M_, K_)).astype(jnp.int32)
    off1 = chb * ch_stride + iy1 * W + ix1
    off2 = chb * ch_stride + iy2 * W + ix2

    zi = jnp.zeros((NB,), jnp.int32)
    iparams = jnp.stack([off1.ravel(), off2.ravel()] + [zi] * 14, axis=1)
    zf = jnp.zeros((NB,), jnp.float32)
    fparams = jnp.stack(
        [w100.ravel(), w101.ravel(), w110.ravel(), w111.ravel(),
         w200.ravel(), w201.ravel(), w210.ravel(), w211.ravel(),
         th.ravel(), apos.ravel(), aneg.ravel()] + [zf] * 5, axis=1)

    Tf = T.reshape(N, Dc, H * W)
    return _fern_sc(Tf, iparams, fparams, N, NB, Hout, Wout, W)


# speedup vs baseline: 1.9598x; 1.8423x over previous
import jax
import jax.numpy as jnp
from jax import lax
from jax.experimental import pallas as pl
from jax.experimental.pallas import tpu as pltpu
from jax.experimental.pallas import tpu_sc as plsc


def _fern_sc(Tf, iparams, fparams, N, NB, Hout, Wout, W):
    ROWS_HALF = 105
    IMG_ROWS = 121
    CH_STRIDE = IMG_ROWS * W
    LOAD_ROWS = 120

    mesh = plsc.VectorSubcoreMesh(core_axis_name="c", subcore_axis_name="s")

    @pl.kernel(
        out_type=jax.ShapeDtypeStruct((N, NB, Hout, Wout), jnp.float32),
        mesh=mesh,
        compiler_params=pltpu.CompilerParams(
            use_tc_tiling_on_sc=False, needs_layout_passes=False),
        scratch_types=[
            pltpu.VMEM((3 * CH_STRIDE,), jnp.float32),
            pltpu.VMEM((NB, 16), jnp.int32),
            pltpu.VMEM((NB, 16), jnp.float32),
            pltpu.VMEM((ROWS_HALF, Wout), jnp.float32),
            pltpu.VMEM((ROWS_HALF, Wout), jnp.float32),
            pltpu.SemaphoreType.DMA,
            pltpu.SemaphoreType.DMA,
        ],
    )
    def k(T_hbm, ipar_hbm, fpar_hbm, out_hbm,
          img_v, ipar_v, fpar_v, obufA, obufB, semA, semB):
        wid = lax.axis_index("s") * 2 + lax.axis_index("c")
        n = wid >> 1
        half = wid & 1
        y0 = half * 104

        pltpu.sync_copy(ipar_hbm, ipar_v)
        pltpu.sync_copy(fpar_hbm, fpar_v)
        for c in range(3):
            pltpu.sync_copy(
                T_hbm.at[n, c, pl.ds(y0 * W, LOAD_ROWS * W)],
                img_v.at[pl.ds(c * CH_STRIDE, LOAD_ROWS * W)])

        def compute_bit(b, obuf):
            ivec = ipar_v[b]
            fvec = fpar_v[b]
            off1 = ivec[0]
            off2 = ivec[1]
            w0 = jnp.full((16,), fvec[0], jnp.float32)
            w1 = jnp.full((16,), fvec[1], jnp.float32)
            w2 = jnp.full((16,), fvec[2], jnp.float32)
            w3 = jnp.full((16,), fvec[3], jnp.float32)
            w4 = jnp.full((16,), fvec[4], jnp.float32)
            w5 = jnp.full((16,), fvec[5], jnp.float32)
            w6 = jnp.full((16,), fvec[6], jnp.float32)
            w7 = jnp.full((16,), fvec[7], jnp.float32)
            thv = jnp.full((16,), fvec[8], jnp.float32)
            apos = jnp.full((16,), fvec[9], jnp.float32)
            aneg = jnp.full((16,), fvec[10], jnp.float32)

            def strip_init(x0):
                ra = off1 + x0
                rb = off2 + x0
                a00 = img_v[pl.ds(ra, 16)]
                a01 = img_v[pl.ds(ra + 1, 16)]
                b00 = img_v[pl.ds(rb, 16)]
                b01 = img_v[pl.ds(rb + 1, 16)]
                ra = ra + W
                rb = rb + W
                a10 = img_v[pl.ds(ra, 16)]
                a11 = img_v[pl.ds(ra + 1, 16)]
                b10 = img_v[pl.ds(rb, 16)]
                b11 = img_v[pl.ds(rb + 1, 16)]
                return (a00, a01, a10, a11, b00, b01, b10, b11, ra, rb)

            def strip_step(cr):
                (a00, a01, a10, a11, b00, b01, b10, b11, ra, rb) = cr
                p1 = (w0 * a00 + w1 * a01) + (w2 * a10 + w3 * a11)
                p2 = (w4 * b00 + w5 * b01) + (w6 * b10 + w7 * b11)
                d = (p1 - p2) - thv
                asel = jnp.where(d >= 0.0, apos, aneg)
                res = jnp.minimum(jnp.maximum(d * asel + 0.5, 0.0), 1.0)
                ra = ra + W
                rb = rb + W
                na0 = img_v[pl.ds(ra, 16)]
                na1 = img_v[pl.ds(ra + 1, 16)]
                nb0 = img_v[pl.ds(rb, 16)]
                nb1 = img_v[pl.ds(rb + 1, 16)]
                return res, (a10, a11, na0, na1, b10, b11, nb0, nb1, ra, rb)

            for jp in range(7):
                xa = 32 * jp
                xb = 193 if jp == 6 else 32 * jp + 16

                def row_body(r, cr):
                    ca, cb = cr
                    res_a, ca = strip_step(ca)
                    res_b, cb = strip_step(cb)
                    obuf[r, pl.ds(xa, 16)] = res_a
                    obuf[r, pl.ds(xb, 16)] = res_b
                    return (ca, cb)

                lax.fori_loop(0, ROWS_HALF, row_body,
                              (strip_init(xa), strip_init(xb)), unroll=3)

        def out_slice(b):
            return out_hbm.at[n, b, pl.ds(y0, ROWS_HALF)]

        def pair_body(g, carry):
            b0 = 2 * g
            b1 = b0 + 1

            @pl.when(g > 0)
            def _():
                pltpu.make_async_copy(obufA, out_slice(b0), semA).wait()

            compute_bit(b0, obufA)
            pltpu.async_copy(obufA, out_slice(b0), semA)

            @pl.when(g > 0)
            def _():
                pltpu.make_async_copy(obufB, out_slice(b1), semB).wait()

            compute_bit(b1, obufB)
            pltpu.async_copy(obufB, out_slice(b1), semB)
            return carry

        lax.fori_loop(0, NB // 2, pair_body, 0)
        pltpu.make_async_copy(obufA, out_slice(NB - 2), semA).wait()
        pltpu.make_async_copy(obufB, out_slice(NB - 1), semB).wait()

    return k(Tf, iparams, fparams)


def kernel(T, dx1, dx2, dy1, dy2, th, amb, channels):
    N, Dc, H, W = T.shape
    M_, K_ = dx1.shape
    NB = M_ * K_
    P_ = 16
    Hout = H - P_ + 1
    Wout = W - P_ + 1
    pm = (P_ - 1) // 2
    ch_stride = (104 + P_ + 1) * W

    def taps(dy, dx):
        fly, flx = jnp.floor(dy), jnp.floor(dx)
        iy = (pm + fly).astype(jnp.int32)
        ix = (pm + flx).astype(jnp.int32)
        fy = jnp.abs(dy - fly)
        fx = jnp.abs(dx - flx)
        w00 = (1.0 - fy) * (1.0 - fx)
        w01 = (1.0 - fy) * fx
        w10 = fy * (1.0 - fx)
        w11 = fy * fx
        return iy, ix, w00, w01, w10, w11

    iy1, ix1, w100, w101, w110, w111 = taps(dy1, dx1)
    iy2, ix2, w200, w201, w210, w211 = taps(dy2, dx2)
    apos = 0.5 / amb[:, 0, :]
    aneg = 0.5 / amb[:, 1, :]
    chb = jnp.broadcast_to(channels[None, :], (M_, K_)).astype(jnp.int32)
    off1 = chb * ch_stride + iy1 * W + ix1
    off2 = chb * ch_stride + iy2 * W + ix2

    zi = jnp.zeros((NB,), jnp.int32)
    iparams = jnp.stack([off1.ravel(), off2.ravel()] + [zi] * 14, axis=1)
    zf = jnp.zeros((NB,), jnp.float32)
    fparams = jnp.stack(
        [w100.ravel(), w101.ravel(), w110.ravel(), w111.ravel(),
         w200.ravel(), w201.ravel(), w210.ravel(), w211.ravel(),
         th.ravel(), apos.ravel(), aneg.ravel()] + [zf] * 5, axis=1)

    Tf = T.reshape(N, Dc, H * W)
    return _fern_sc(Tf, iparams, fparams, N, NB, Hout, Wout, W)
